# baseline (speedup 1.0000x reference)
import jax
import jax.numpy as jnp
from jax import lax
from jax.experimental import pallas as pl
from jax.experimental.pallas import tpu as pltpu
from jax.experimental.pallas import tpu_sc as plsc

N = 10000
NPAD = 10240
D = 256
DH = 128
E = 160000
CH = 128
NCH = 80
EPAD = 16 * NCH * CH
RT = NPAD // 16
RCH = RT // CH
NLANE = 16


def _sc_body(u1, u2, fi, e1sg, e1dg, e2sg, e2dg, e1s, e1d, e2s, e2d,
             cj1, cj2, cjg, cig, cic1, cic2,
             outc1, outc2, outg, s1, s2, sg,
             acc, idx_g, idx_s, rows, vecbuf, sem):
    c = lax.axis_index("c")
    t = lax.axis_index("s")
    base = t * RT

    def scale_rows(off):
        def sgrp(g, carry):
            sv = vecbuf[pl.ds(off + g * NLANE, NLANE)]
            for r in range(NLANE):
                s = sv[r]
                e = g * NLANE + r
                for q in range(DH // NLANE):
                    sl = pl.ds(q * NLANE, NLANE)
                    rows[e, sl] = rows[e, sl] * s
            return carry
        lax.fori_loop(0, CH // NLANE, sgrp, None)

    def prescale(table, cj_r, s_out):
        pltpu.sync_copy(cj_r.at[t], vecbuf)

        def pm(m, carry):
            r0 = base + m * CH
            pltpu.sync_copy(table.at[pl.ds(r0, CH), pl.ds(c * DH, DH)], rows)
            scale_rows(m * CH)
            pltpu.sync_copy(rows, s_out.at[pl.ds(c * NPAD + r0, CH)])
            return carry
        lax.fori_loop(0, RCH, pm, None)

    def zero_acc():
        def zrow(e, carry):
            for q in range(DH // NLANE):
                rows[e, pl.ds(q * NLANE, NLANE)] = jnp.zeros((NLANE,), jnp.float32)
            return carry
        lax.fori_loop(0, CH, zrow, None)

        def zm(m, carry):
            pltpu.sync_copy(rows, acc.at[pl.ds(base + m * CH, CH)])
            return carry
        lax.fori_loop(0, RCH, zm, None)

    def accumulate(g_edges, s_edges, s_tab):
        pltpu.sync_copy(g_edges.at[c, t], idx_g)
        pltpu.sync_copy(s_edges.at[t], idx_s)

        def cb(j, carry):
            pltpu.async_copy(s_tab.at[idx_g.at[j]], rows, sem).wait()
            pltpu.sync_copy(rows, acc.at[idx_s.at[j]], add=True)
            return carry
        lax.fori_loop(0, NCH, cb, None)

    def readout(ci_r, out_ref):
        pltpu.sync_copy(ci_r.at[t], vecbuf)

        def rm(m, carry):
            r0 = base + m * CH
            pltpu.sync_copy(acc.at[pl.ds(r0, CH)], rows)
            scale_rows(m * CH)
            pltpu.sync_copy(rows, out_ref.at[pl.ds(r0, CH), pl.ds(c * DH, DH)])
            return carry
        lax.fori_loop(0, RCH, rm, None)

    prescale(u1, cj1, s1)
    prescale(u2, cj2, s2)
    prescale(fi, cjg, sg)
    zero_acc()
    plsc.subcore_barrier()
    accumulate(e1sg, e1d, s1)
    accumulate(e2sg, e2d, s2)
    plsc.subcore_barrier()
    readout(cig, outg)
    plsc.subcore_barrier()
    zero_acc()
    plsc.subcore_barrier()
    accumulate(e1dg, e1s, sg)
    plsc.subcore_barrier()
    readout(cic1, outc1)
    plsc.subcore_barrier()
    zero_acc()
    plsc.subcore_barrier()
    accumulate(e2dg, e2s, sg)
    plsc.subcore_barrier()
    readout(cic2, outc2)


def _make_sc_kernel(interpret=False):
    mesh = plsc.VectorSubcoreMesh(core_axis_name="c", subcore_axis_name="s")
    f32 = jnp.float32
    return pl.kernel(
        _sc_body,
        out_type=(
            jax.ShapeDtypeStruct((NPAD, D), f32),
            jax.ShapeDtypeStruct((NPAD, D), f32),
            jax.ShapeDtypeStruct((NPAD, D), f32),
            jax.ShapeDtypeStruct((2 * NPAD, DH), f32),
            jax.ShapeDtypeStruct((2 * NPAD, DH), f32),
            jax.ShapeDtypeStruct((2 * NPAD, DH), f32),
        ),
        mesh=mesh,
        scratch_types=[
            pltpu.VMEM_SHARED((NPAD, DH), f32),
            pltpu.VMEM((NCH, CH), jnp.int32),
            pltpu.VMEM((NCH, CH), jnp.int32),
            pltpu.VMEM((CH, DH), f32),
            pltpu.VMEM((RT,), f32),
            pltpu.SemaphoreType.DMA,
        ],
        interpret=interpret,
    )


_sc_kernel = _make_sc_kernel()


def kernel(ufeats1, ufeats2, ifeats, edges_1, edges_2, cj_cell1, ci_cell1,
           cj_cell2, ci_cell2, cj_gene, ci_gene):
    f32 = jnp.float32

    def padtab(x):
        return jnp.pad(x.astype(f32), ((0, NPAD - N), (0, 0)))

    def padvec(x, scale=None):
        v = jnp.pad(x.astype(f32)[:, 0], (0, NPAD - N))
        if scale is not None:
            v = v * scale
        return v.reshape(16, RT)

    def pad_edges(e):
        ep = jnp.pad(e.astype(jnp.int32), ((0, 0), (0, EPAD - E)),
                     constant_values=N)
        src = ep[0].reshape(16, NCH, CH)
        dst = ep[1].reshape(16, NCH, CH)
        srcg = jnp.stack([src, src + NPAD])
        dstg = jnp.stack([dst, dst + NPAD])
        return src, dst, srcg, dstg

    u1 = padtab(ufeats1)
    u2 = padtab(ufeats2)
    fi = padtab(ifeats)
    e1s, e1d, e1sg, e1dg = pad_edges(edges_1)
    e2s, e2d, e2sg, e2dg = pad_edges(edges_2)
    cj1 = padvec(cj_cell1)
    cj2 = padvec(cj_cell2)
    cjg = padvec(cj_gene)
    cig = padvec(ci_gene, scale=0.5)
    cic1 = padvec(ci_cell1)
    cic2 = padvec(ci_cell2)

    outc1, outc2, outg, _, _, _ = _sc_kernel(
        u1, u2, fi, e1sg, e1dg, e2sg, e2dg, e1s, e1d, e2s, e2d,
        cj1, cj2, cjg, cig, cic1, cic2)
    return (outc1[:N], outc2[:N], outg[:N])


# speedup vs baseline: 2.3267x; 2.3267x over previous
import jax
import jax.numpy as jnp
from jax import lax
from jax.experimental import pallas as pl
from jax.experimental.pallas import tpu as pltpu
from jax.experimental.pallas import tpu_sc as plsc

N = 10000
NPAD = 10240
D = 256
DH = 128
E = 160000
CH = 128
NCH = 80
EPAD = 16 * NCH * CH
GW = 16
NW = NCH // GW
RT = NPAD // 16
RCH = RT // CH
NLANE = 16


def _sc_body(u1, u2, fi, e1sg, e1dg, e2sg, e2dg, e1s, e1d, e2s, e2d,
             cj1, cj2, cjg, cig, cic1, cic2,
             outc1, outc2, outg, s1, s2, sg,
             acc, idx_g, idx_s, rows, rows2, vecbuf, sem, ssem0, ssem1):
    c = lax.axis_index("c")
    t = lax.axis_index("s")
    base = t * RT

    def scale_rows(off):
        def sgrp(g, carry):
            sv = vecbuf[pl.ds(off + g * NLANE, NLANE)]
            for r in range(NLANE):
                s = sv[r]
                e = g * NLANE + r
                for q in range(DH // NLANE):
                    sl = pl.ds(q * NLANE, NLANE)
                    rows[e, sl] = rows[e, sl] * s
            return carry
        lax.fori_loop(0, CH // NLANE, sgrp, None)

    def prescale(table, cj_r, s_out):
        pltpu.sync_copy(cj_r.at[t], vecbuf)

        def pm(m, carry):
            r0 = base + m * CH
            pltpu.sync_copy(table.at[pl.ds(r0, CH), pl.ds(c * DH, DH)], rows)
            scale_rows(m * CH)
            pltpu.sync_copy(rows, s_out.at[pl.ds(c * NPAD + r0, CH)])
            return carry
        lax.fori_loop(0, RCH, pm, None)

    def zero_acc():
        def zrow(e, carry):
            for q in range(DH // NLANE):
                rows[e, pl.ds(q * NLANE, NLANE)] = jnp.zeros((NLANE,), jnp.float32)
            return carry
        lax.fori_loop(0, CH, zrow, None)

        def zm(m, carry):
            pltpu.sync_copy(rows, acc.at[pl.ds(base + m * CH, CH)])
            return carry
        lax.fori_loop(0, RCH, zm, None)

    def accumulate(g_edges, s_edges, s_tab):
        rbufs = (rows, rows2)
        ssems = (ssem0, ssem1)

        def wloop(w, carry):
            pltpu.sync_copy(g_edges.at[c, t, pl.ds(w * GW, GW)], idx_g)
            pltpu.sync_copy(s_edges.at[t, pl.ds(w * GW, GW)], idx_s)

            def cb(j, carry2):
                for b in range(2):
                    k = j * 2 + b
                    g = w * GW + k

                    @pl.when(g >= 2)
                    def _():
                        pltpu.make_async_copy(s_tab.at[pl.ds(0, CH)],
                                              rbufs[b], ssems[b]).wait()
                    pltpu.async_copy(s_tab.at[idx_g.at[k]], rbufs[b],
                                     sem).wait()
                    pltpu.async_copy(rbufs[b], acc.at[idx_s.at[k]], ssems[b],
                                     add=True)
                return carry2
            lax.fori_loop(0, GW // 2, cb, None)
            return carry
        lax.fori_loop(0, NW, wloop, None)
        for b in range(2):
            pltpu.make_async_copy(s_tab.at[pl.ds(0, CH)], rbufs[b],
                                  ssems[b]).wait()

    def readout(ci_r, out_ref):
        pltpu.sync_copy(ci_r.at[t], vecbuf)

        def rm(m, carry):
            r0 = base + m * CH
            pltpu.sync_copy(acc.at[pl.ds(r0, CH)], rows)
            scale_rows(m * CH)
            pltpu.sync_copy(rows, out_ref.at[pl.ds(r0, CH), pl.ds(c * DH, DH)])
            return carry
        lax.fori_loop(0, RCH, rm, None)

    prescale(u1, cj1, s1)
    prescale(u2, cj2, s2)
    prescale(fi, cjg, sg)
    zero_acc()
    plsc.subcore_barrier()
    accumulate(e1sg, e1d, s1)
    accumulate(e2sg, e2d, s2)
    plsc.subcore_barrier()
    readout(cig, outg)
    plsc.subcore_barrier()
    zero_acc()
    plsc.subcore_barrier()
    accumulate(e1dg, e1s, sg)
    plsc.subcore_barrier()
    readout(cic1, outc1)
    plsc.subcore_barrier()
    zero_acc()
    plsc.subcore_barrier()
    accumulate(e2dg, e2s, sg)
    plsc.subcore_barrier()
    readout(cic2, outc2)


def _make_sc_kernel(interpret=False):
    mesh = plsc.VectorSubcoreMesh(core_axis_name="c", subcore_axis_name="s")
    f32 = jnp.float32
    return pl.kernel(
        _sc_body,
        out_type=(
            jax.ShapeDtypeStruct((NPAD, D), f32),
            jax.ShapeDtypeStruct((NPAD, D), f32),
            jax.ShapeDtypeStruct((NPAD, D), f32),
            jax.ShapeDtypeStruct((2 * NPAD, DH), f32),
            jax.ShapeDtypeStruct((2 * NPAD, DH), f32),
            jax.ShapeDtypeStruct((2 * NPAD, DH), f32),
        ),
        mesh=mesh,
        scratch_types=[
            pltpu.VMEM_SHARED((NPAD, DH), f32),
            pltpu.VMEM((GW, CH), jnp.int32),
            pltpu.VMEM((GW, CH), jnp.int32),
            pltpu.VMEM((CH, DH), f32),
            pltpu.VMEM((CH, DH), f32),
            pltpu.VMEM((RT,), f32),
            pltpu.SemaphoreType.DMA,
            pltpu.SemaphoreType.DMA,
            pltpu.SemaphoreType.DMA,
        ],
        interpret=interpret,
    )


_sc_kernel = _make_sc_kernel()


def kernel(ufeats1, ufeats2, ifeats, edges_1, edges_2, cj_cell1, ci_cell1,
           cj_cell2, ci_cell2, cj_gene, ci_gene):
    f32 = jnp.float32

    def padtab(x):
        return jnp.pad(x.astype(f32), ((0, NPAD - N), (0, 0)))

    def padvec(x, scale=None):
        v = jnp.pad(x.astype(f32)[:, 0], (0, NPAD - N))
        if scale is not None:
            v = v * scale
        return v.reshape(16, RT)

    def pad_edges(e):
        fill = N + (jnp.arange(EPAD - E, dtype=jnp.int32) % (NPAD - N))
        fill2 = jnp.stack([fill, fill])
        ep = jnp.concatenate([e.astype(jnp.int32), fill2], axis=1)
        src = ep[0].reshape(16, NCH, CH)
        dst = ep[1].reshape(16, NCH, CH)
        srcg = jnp.stack([src, src + NPAD])
        dstg = jnp.stack([dst, dst + NPAD])
        return src, dst, srcg, dstg

    u1 = padtab(ufeats1)
    u2 = padtab(ufeats2)
    fi = padtab(ifeats)
    e1s, e1d, e1sg, e1dg = pad_edges(edges_1)
    e2s, e2d, e2sg, e2dg = pad_edges(edges_2)
    cj1 = padvec(cj_cell1)
    cj2 = padvec(cj_cell2)
    cjg = padvec(cj_gene)
    cig = padvec(ci_gene, scale=0.5)
    cic1 = padvec(ci_cell1)
    cic2 = padvec(ci_cell2)

    outc1, outc2, outg, _, _, _ = _sc_kernel(
        u1, u2, fi, e1sg, e1dg, e2sg, e2dg, e1s, e1d, e2s, e2d,
        cj1, cj2, cjg, cig, cic1, cic2)
    return (outc1[:N], outc2[:N], outg[:N])
